# baseline (speedup 1.0000x reference)
import functools

import jax
import jax.numpy as jnp
from jax import lax
from jax.experimental import pallas as pl
from jax.experimental.pallas import tpu as pltpu
from jax.experimental.pallas import tpu_sc as plsc


def _make_sc_part(B, D, H, dtype):
    info = plsc.get_sparse_core_info()
    nw = info.num_cores * info.num_subcores
    rows = B - H
    r_per_w = rows // nw
    assert r_per_w * nw == rows

    mesh = plsc.VectorSubcoreMesh(core_axis_name="c", subcore_axis_name="s")

    @functools.partial(
        pl.kernel,
        out_type=jax.ShapeDtypeStruct((rows, D), dtype),
        mesh=mesh,
        scratch_types=[
            pltpu.VMEM((r_per_w, D), dtype),
            pltpu.SemaphoreType.DMA,
            pltpu.SemaphoreType.DMA,
        ],
    )
    def sc_part(item_hbm, out_hbm, rows_v, sem_in, sem_out):
        wid = lax.axis_index("s") * info.num_cores + lax.axis_index("c")
        pltpu.async_copy(
            item_hbm.at[pl.ds(H + wid * r_per_w, r_per_w)], rows_v, sem_in
        ).wait()
        pltpu.async_copy(
            rows_v, out_hbm.at[pl.ds(wid * r_per_w, r_per_w)], sem_out
        ).wait()

    return sc_part


def _tc_copy_body(item_ref, out_ref):
    out_ref[...] = item_ref[...]


def _make_tc_part(B, D, H, dtype, blk=512):
    assert H % blk == 0
    return pl.pallas_call(
        _tc_copy_body,
        grid=(H // blk,),
        in_specs=[pl.BlockSpec((blk, D), lambda i: (i, 0))],
        out_specs=pl.BlockSpec((blk, D), lambda i: (i, 0)),
        out_shape=jax.ShapeDtypeStruct((H, D), dtype),
    )


def kernel(item, out, order_indices):
    B, D = item.shape
    H = 9728
    sc_tail = _make_sc_part(B, D, H, item.dtype)(item)
    tc_head = _make_tc_part(B, D, H, item.dtype)(item)
    return jnp.concatenate([tc_head, sc_tail], axis=0)


# speedup vs baseline: 3.9606x; 3.9606x over previous
import jax
import jax.numpy as jnp
from jax.experimental import pallas as pl


def _tc_copy_body(item_ref, out_ref):
    out_ref[...] = item_ref[...]


def kernel(item, out, order_indices):
    B, D = item.shape
    blk = 2048
    return pl.pallas_call(
        _tc_copy_body,
        grid=(B // blk,),
        in_specs=[pl.BlockSpec((blk, D), lambda i: (i, 0))],
        out_specs=pl.BlockSpec((blk, D), lambda i: (i, 0)),
        out_shape=jax.ShapeDtypeStruct((B, D), item.dtype),
    )(item)
